# baseline (speedup 1.0000x reference)
import functools

import jax
import jax.numpy as jnp
from jax import lax
from jax.experimental import pallas as pl
from jax.experimental.pallas import tpu as pltpu
from jax.experimental.pallas import tpu_sc as plsc

PRE_SEQ_LEN = 128
BATCH = 16
EMB_DIM = 18432
N_ROWS = BATCH * PRE_SEQ_LEN
N_CORES = 2
ROWS_PER_C = N_ROWS // N_CORES

_MESH = plsc.ScalarSubcoreMesh(axis_name="core", num_cores=N_CORES)


@functools.partial(
    pl.kernel,
    mesh=_MESH,
    out_type=jax.ShapeDtypeStruct((N_ROWS, EMB_DIM), jnp.float32),
    scratch_types=[
        pltpu.SMEM((ROWS_PER_C,), jnp.int32),
        pltpu.SemaphoreType.DMA,
        pltpu.SemaphoreType.DMA,
    ],
)
def _sc_gather(tbl_hbm, idx_hbm, out_hbm, idx_s, sem_i, sem):
    core = lax.axis_index("core")
    base = core * ROWS_PER_C

    cp_i = pltpu.make_async_copy(idx_hbm.at[pl.ds(base, ROWS_PER_C)], idx_s, sem_i)
    cp_i.start()
    cp_i.wait()

    def row(g):
        idx = idx_s[g]
        return pltpu.make_async_copy(
            tbl_hbm.at[pl.ds(idx, 1)],
            out_hbm.at[pl.ds(base + g, 1)],
            sem,
        )

    @pl.loop(0, ROWS_PER_C)
    def _(g):
        row(g).start()

    @pl.loop(0, ROWS_PER_C)
    def _(g):
        row(g).wait()


def kernel(prefix, embedding_table):
    out = _sc_gather(embedding_table, prefix.reshape(N_ROWS))
    return out.reshape(BATCH, PRE_SEQ_LEN, EMB_DIM)


# speedup vs baseline: 36.4419x; 36.4419x over previous
import functools

import jax
import jax.numpy as jnp
from jax import lax
from jax.experimental import pallas as pl
from jax.experimental.pallas import tpu as pltpu
from jax.experimental.pallas import tpu_sc as plsc

PRE_SEQ_LEN = 128
BATCH = 16
EMB_DIM = 18432
N_ROWS = BATCH * PRE_SEQ_LEN
NW = 32
ROWS_PER_W = N_ROWS // NW
NBUF = 4

_MESH = plsc.VectorSubcoreMesh(core_axis_name="core", subcore_axis_name="subcore")


@functools.partial(
    pl.kernel,
    mesh=_MESH,
    out_type=jax.ShapeDtypeStruct((N_ROWS, EMB_DIM), jnp.float32),
    scratch_types=[
        pltpu.VMEM((1, 128), jnp.int32),
        pltpu.VMEM((NBUF, 1, EMB_DIM), jnp.float32),
        pltpu.SemaphoreType.DMA,
        pltpu.SemaphoreType.DMA((NBUF,)),
        pltpu.SemaphoreType.DMA((NBUF,)),
    ],
)
def _sc_gather(tbl_hbm, idx_hbm, out_hbm, idx_v, bufs, sem_i, sem_g, sem_o):
    wid = lax.axis_index("subcore") * 2 + lax.axis_index("core")
    base = wid * ROWS_PER_W
    half = (wid % 2) * ROWS_PER_W

    cp_i = pltpu.make_async_copy(idx_hbm.at[pl.ds(wid // 2, 1)], idx_v, sem_i)
    cp_i.start()
    cp_i.wait()

    def gather(g, b):
        return pltpu.make_async_copy(
            tbl_hbm.at[idx_v.at[0, pl.ds(half + g, 1)]], bufs.at[b], sem_g.at[b]
        )

    def put(g, b):
        return pltpu.make_async_copy(
            bufs.at[b], out_hbm.at[pl.ds(base + g, 1)], sem_o.at[b]
        )

    for b in range(NBUF - 1):
        gather(b, b).start()

    @pl.loop(0, ROWS_PER_W, step=NBUF)
    def _(g0):
        for j in range(NBUF):
            g = g0 + j
            gather(g, j).wait()
            put(g, j).start()
            nxt = g + NBUF - 1
            bn = (j + NBUF - 1) % NBUF
            prev = g - 1

            @pl.when(nxt < ROWS_PER_W)
            def _():
                @pl.when(prev >= 0)
                def _():
                    put(prev, bn).wait()

                gather(nxt, bn).start()

    for j in range(NBUF):
        put(ROWS_PER_W - NBUF + j, j).wait()


def kernel(prefix, embedding_table):
    out = _sc_gather(embedding_table, prefix)
    return out.reshape(BATCH, PRE_SEQ_LEN, EMB_DIM)
